# Initial kernel scaffold; baseline (speedup 1.0000x reference)
import jax
import jax.numpy as jnp
from jax.experimental import pallas as pl


def kernel(x, edge_index, Q0_w, Q0_b, W0_w, W0_b, Q1_w, Q1_b, W1_w, W1_b, G_w, G_b):
    raise NotImplementedError("write your pallas kernel here")


# speedup vs baseline: 6.0238x; 6.0238x over previous
import functools

import jax
import jax.numpy as jnp
from jax import lax
from jax.experimental import pallas as pl
from jax.experimental.pallas import tpu as pltpu
from jax.experimental.pallas import tpu_sc as plsc

N = 10000
D = 128
H = 128
E = 320000
CH = 128
NCHUNKS = E // CH
NW = 32
NSUB = 16
RPS = 624
DEGROWS = 10240


def _sc_aggregate(m, src, dst, with_deg):
    mesh = plsc.VectorSubcoreMesh(core_axis_name="c", subcore_axis_name="s")

    out_type = [jax.ShapeDtypeStruct((2, N, H), jnp.float32)]
    if with_deg:
        out_type.append(jax.ShapeDtypeStruct((2, DEGROWS), jnp.float32))

    scratch = [
        pltpu.VMEM((1, CH), jnp.int32),
        pltpu.VMEM((1, CH), jnp.int32),
        pltpu.VMEM((CH, H), jnp.float32),
        pltpu.VMEM((16, H), jnp.float32),
        pltpu.VMEM_SHARED((N, H), jnp.float32),
        pltpu.SemaphoreType.DMA,
    ]
    if with_deg:
        scratch += [
            pltpu.VMEM((CH,), jnp.float32),
            pltpu.VMEM((640,), jnp.float32),
            pltpu.VMEM_SHARED((DEGROWS,), jnp.float32),
        ]

    def body(m_hbm, src_hbm, dst_hbm, *rest):
        if with_deg:
            (parts_out, degp_out, sidx, didx, rows, zrow, acc, gsem,
             ones_b, dz, dacc) = rest
        else:
            (parts_out, sidx, didx, rows, zrow, acc, gsem) = rest

        cid = lax.axis_index("c")
        sid = lax.axis_index("s")
        wid = sid * 2 + cid

        def zfill(i, _):
            for j in range(H // 16):
                zrow[i, pl.ds(j * 16, 16)] = jnp.zeros((16,), jnp.float32)
            return 0
        lax.fori_loop(0, 16, zfill, 0)

        if with_deg:
            def dzfill(i, _):
                dz[pl.ds(i * 16, 16)] = jnp.zeros((16,), jnp.float32)
                return 0
            lax.fori_loop(0, 640 // 16, dzfill, 0)

            def onesfill(i, _):
                ones_b[pl.ds(i * 16, 16)] = jnp.ones((16,), jnp.float32)
                return 0
            lax.fori_loop(0, CH // 16, onesfill, 0)

        nz = (RPS // 16) + jnp.where(sid == NSUB - 1, 1, 0)

        def zcopy(k, _):
            pltpu.sync_copy(zrow, acc.at[pl.ds(sid * RPS + k * 16, 16)])
            return 0
        lax.fori_loop(0, nz, zcopy, 0)
        if with_deg:
            pltpu.sync_copy(dz, dacc.at[pl.ds(sid * 640, 640)])
        plsc.subcore_barrier()

        extra = NCHUNKS - (NCHUNKS // NW) * NW
        nch = NCHUNKS // NW + jnp.where(wid < extra, 1, 0)
        base = wid * (NCHUNKS // NW) + jnp.minimum(wid, extra)

        def chunk_body(i, _):
            c = (base + i) * CH
            pltpu.sync_copy(src_hbm.at[pl.ds(c, CH)], sidx.at[0])
            pltpu.sync_copy(dst_hbm.at[pl.ds(c, CH)], didx.at[0])
            pltpu.async_copy(m_hbm.at[sidx.at[0]], rows, gsem).wait()
            pltpu.sync_copy(rows, acc.at[didx.at[0]], add=True)
            if with_deg:
                pltpu.sync_copy(ones_b, dacc.at[didx.at[0]], add=True)
            return 0
        lax.fori_loop(0, nch, chunk_body, 0)

        plsc.subcore_barrier()

        r0 = sid * RPS
        pltpu.sync_copy(acc.at[pl.ds(r0, RPS)],
                        parts_out.at[cid, pl.ds(r0, RPS)])

        @pl.when(sid == NSUB - 1)
        def _tail():
            pltpu.sync_copy(acc.at[pl.ds(NSUB * RPS, N - NSUB * RPS)],
                            parts_out.at[cid, pl.ds(NSUB * RPS, N - NSUB * RPS)])
        if with_deg:
            d0 = sid * 640
            pltpu.sync_copy(dacc.at[pl.ds(d0, 640)],
                            degp_out.at[cid, pl.ds(d0, 640)])

    fn = pl.kernel(body, out_type=out_type, mesh=mesh, scratch_types=scratch)
    return fn(m, src, dst)


def _tc_linear_relu(x, w, b):
    n, k = x.shape
    m = w.shape[1]
    tn = 1000

    def body(x_ref, w_ref, b_ref, o_ref):
        o_ref[...] = jax.nn.relu(
            jnp.dot(x_ref[...], w_ref[...],
                    preferred_element_type=jnp.float32,
                    precision=lax.Precision.HIGHEST) + b_ref[...])

    return pl.pallas_call(
        body,
        grid=(n // tn,),
        in_specs=[
            pl.BlockSpec((tn, k), lambda i: (i, 0)),
            pl.BlockSpec((k, m), lambda i: (0, 0)),
            pl.BlockSpec((1, m), lambda i: (0, 0)),
        ],
        out_specs=pl.BlockSpec((tn, m), lambda i: (i, 0)),
        out_shape=jax.ShapeDtypeStruct((n, m), jnp.float32),
    )(x, w, b.reshape(1, -1))


def _tc_combine(h, parts, degp, w_h, w_a, w_b, m_w, m_b):
    tn = 1000

    def body(h_ref, p_ref, d_ref, wh_ref, wa_ref, wb_ref, mw_ref, mb_ref,
             hn_ref, o2_ref):
        deg = d_ref[0, :, :1] + d_ref[1, :, :1]
        agg = (p_ref[0] + p_ref[1]) / jnp.maximum(deg, 1.0)
        z = jax.nn.relu(
            jnp.dot(h_ref[...], wh_ref[...],
                    preferred_element_type=jnp.float32,
                    precision=lax.Precision.HIGHEST)
            + jnp.dot(agg, wa_ref[...],
                      preferred_element_type=jnp.float32,
                      precision=lax.Precision.HIGHEST)
            + wb_ref[...])
        nrm = jnp.sqrt(jnp.sum(z * z, axis=-1, keepdims=True))
        hn = z / jnp.maximum(nrm, 1e-6)
        hn_ref[...] = hn
        o2_ref[...] = jax.nn.relu(
            jnp.dot(hn, mw_ref[...],
                    preferred_element_type=jnp.float32,
                    precision=lax.Precision.HIGHEST) + mb_ref[...])

    return pl.pallas_call(
        body,
        grid=(N // tn,),
        in_specs=[
            pl.BlockSpec((tn, D), lambda i: (i, 0)),
            pl.BlockSpec((2, tn, H), lambda i: (0, i, 0)),
            pl.BlockSpec((2, tn, 1), lambda i: (0, i, 0)),
            pl.BlockSpec((D, D), lambda i: (0, 0)),
            pl.BlockSpec((H, D), lambda i: (0, 0)),
            pl.BlockSpec((1, D), lambda i: (0, 0)),
            pl.BlockSpec((D, D), lambda i: (0, 0)),
            pl.BlockSpec((1, D), lambda i: (0, 0)),
        ],
        out_specs=[
            pl.BlockSpec((tn, D), lambda i: (i, 0)),
            pl.BlockSpec((tn, D), lambda i: (i, 0)),
        ],
        out_shape=[
            jax.ShapeDtypeStruct((N, D), jnp.float32),
            jax.ShapeDtypeStruct((N, D), jnp.float32),
        ],
    )(h, parts, degp, w_h, w_a, w_b.reshape(1, -1), m_w, m_b.reshape(1, -1))


def kernel(x, edge_index, Q0_w, Q0_b, W0_w, W0_b, Q1_w, Q1_b, W1_w, W1_b, G_w, G_b):
    src = edge_index[0]
    dst = edge_index[1]

    m0 = _tc_linear_relu(x, Q0_w, Q0_b)
    parts0, degp = _sc_aggregate(m0, src, dst, with_deg=True)
    degp = degp[:, :N].reshape(2, N, 1)

    h1, m1 = _tc_combine(x, parts0, degp, W0_w[:D], W0_w[D:], W0_b, Q1_w, Q1_b)

    parts1 = _sc_aggregate(m1, src, dst, with_deg=False)[0]
    _, out = _tc_combine(h1, parts1, degp, W1_w[:D], W1_w[D:], W1_b, G_w, G_b)
    return out


# speedup vs baseline: 9.1768x; 1.5234x over previous
import functools

import jax
import jax.numpy as jnp
from jax import lax
from jax.experimental import pallas as pl
from jax.experimental.pallas import tpu as pltpu
from jax.experimental.pallas import tpu_sc as plsc

N = 10000
D = 128
H = 128
E = 320000
CH = 128
NCHUNKS = E // CH
NW = 32
NSUB = 16
RPS = 624
DEGROWS = 10240


def _sc_aggregate(m, src, dst, with_deg):
    mesh = plsc.VectorSubcoreMesh(core_axis_name="c", subcore_axis_name="s")

    out_type = [jax.ShapeDtypeStruct((2, N, H), jnp.float32)]
    if with_deg:
        out_type.append(jax.ShapeDtypeStruct((2, DEGROWS), jnp.float32))

    scratch = [
        pltpu.VMEM((2, CH), jnp.int32),
        pltpu.VMEM((2, CH), jnp.int32),
        pltpu.VMEM((CH, H), jnp.float32),
        pltpu.VMEM((CH, H), jnp.float32),
        pltpu.VMEM((16, H), jnp.float32),
        pltpu.VMEM_SHARED((N, H), jnp.float32),
        pltpu.SemaphoreType.DMA,
        pltpu.SemaphoreType.DMA,
        pltpu.SemaphoreType.DMA,
        pltpu.SemaphoreType.DMA,
        pltpu.SemaphoreType.DMA,
        pltpu.SemaphoreType.DMA,
    ]
    if with_deg:
        scratch += [
            pltpu.VMEM((CH,), jnp.float32),
            pltpu.VMEM((640,), jnp.float32),
            pltpu.VMEM_SHARED((DEGROWS,), jnp.float32),
            pltpu.SemaphoreType.DMA,
        ]

    def body(m_hbm, src_hbm, dst_hbm, *rest):
        if with_deg:
            (parts_out, degp_out, sidx, didx, rows0, rows1, zrow, acc,
             si0, si1, di0, di1, sg0, sg1, ones_b, dz, dacc, sd) = rest
        else:
            (parts_out, sidx, didx, rows0, rows1, zrow, acc,
             si0, si1, di0, di1, sg0, sg1) = rest
        rows = (rows0, rows1)
        si = (si0, si1)
        di = (di0, di1)
        sg = (sg0, sg1)

        cid = lax.axis_index("c")
        sid = lax.axis_index("s")
        wid = sid * 2 + cid

        def zfill(i, _):
            for j in range(H // 16):
                zrow[i, pl.ds(j * 16, 16)] = jnp.zeros((16,), jnp.float32)
            return 0
        lax.fori_loop(0, 16, zfill, 0)

        if with_deg:
            def dzfill(i, _):
                dz[pl.ds(i * 16, 16)] = jnp.zeros((16,), jnp.float32)
                return 0
            lax.fori_loop(0, 640 // 16, dzfill, 0)

            def onesfill(i, _):
                ones_b[pl.ds(i * 16, 16)] = jnp.ones((16,), jnp.float32)
                return 0
            lax.fori_loop(0, CH // 16, onesfill, 0)

        nz = (RPS // 16) + jnp.where(sid == NSUB - 1, 1, 0)

        def zcopy(k, _):
            pltpu.sync_copy(zrow, acc.at[pl.ds(sid * RPS + k * 16, 16)])
            return 0
        lax.fori_loop(0, nz, zcopy, 0)
        if with_deg:
            pltpu.sync_copy(dz, dacc.at[pl.ds(sid * 640, 640)])
        plsc.subcore_barrier()

        extra = NCHUNKS - (NCHUNKS // NW) * NW
        nch = NCHUNKS // NW + jnp.where(wid < extra, 1, 0)
        base = wid * (NCHUNKS // NW) + jnp.minimum(wid, extra)
        nchp = NCHUNKS // NW + 2

        def issue_idx(c, b):
            pltpu.async_copy(src_hbm.at[pl.ds((base + c) * CH, CH)],
                             sidx.at[b], si[b])
            pltpu.async_copy(dst_hbm.at[pl.ds((base + c) * CH, CH)],
                             didx.at[b], di[b])

        def wait_idx(b):
            pltpu.make_async_copy(src_hbm.at[pl.ds(0, CH)], sidx.at[b],
                                  si[b]).wait()
            pltpu.make_async_copy(dst_hbm.at[pl.ds(0, CH)], didx.at[b],
                                  di[b]).wait()

        issue_idx(0, 0)
        issue_idx(1, 1)
        wait_idx(0)
        pltpu.async_copy(m_hbm.at[sidx.at[0]], rows[0], sg[0])

        def pair_body(i, _):
            for b in (0, 1):
                b1 = 1 - b
                c = 2 * i + b
                wait_idx(b1)
                pltpu.async_copy(m_hbm.at[sidx.at[b1]], rows[b1], sg[b1])
                if with_deg:
                    @pl.when(c < nch)
                    def _fire_deg():
                        pltpu.async_copy(ones_b, dacc.at[didx.at[b]], sd,
                                         add=True)
                pltpu.make_async_copy(m_hbm.at[sidx.at[b]], rows[b],
                                      sg[b]).wait()
                if with_deg:
                    @pl.when(c < nch)
                    def _wait_deg():
                        pltpu.make_async_copy(ones_b, dacc.at[didx.at[b]],
                                              sd).wait()

                @pl.when(c < nch)
                def _scatter():
                    pltpu.sync_copy(rows[b], acc.at[didx.at[b]], add=True)
                issue_idx(c + 2, b)
            return 0
        lax.fori_loop(0, nchp // 2, pair_body, 0)

        pltpu.make_async_copy(m_hbm.at[sidx.at[0]], rows[0], sg[0]).wait()
        wait_idx(1)

        plsc.subcore_barrier()

        r0 = sid * RPS
        pltpu.sync_copy(acc.at[pl.ds(r0, RPS)],
                        parts_out.at[cid, pl.ds(r0, RPS)])

        @pl.when(sid == NSUB - 1)
        def _tail():
            pltpu.sync_copy(acc.at[pl.ds(NSUB * RPS, N - NSUB * RPS)],
                            parts_out.at[cid, pl.ds(NSUB * RPS, N - NSUB * RPS)])
        if with_deg:
            d0 = sid * 640
            pltpu.sync_copy(dacc.at[pl.ds(d0, 640)],
                            degp_out.at[cid, pl.ds(d0, 640)])

    fn = pl.kernel(body, out_type=out_type, mesh=mesh, scratch_types=scratch)
    return fn(m, src, dst)


def _tc_linear_relu(x, w, b):
    n, k = x.shape
    m = w.shape[1]
    tn = 1000

    def body(x_ref, w_ref, b_ref, o_ref):
        o_ref[...] = jax.nn.relu(
            jnp.dot(x_ref[...], w_ref[...],
                    preferred_element_type=jnp.float32,
                    precision=lax.Precision.HIGHEST) + b_ref[...])

    return pl.pallas_call(
        body,
        grid=(n // tn,),
        in_specs=[
            pl.BlockSpec((tn, k), lambda i: (i, 0)),
            pl.BlockSpec((k, m), lambda i: (0, 0)),
            pl.BlockSpec((1, m), lambda i: (0, 0)),
        ],
        out_specs=pl.BlockSpec((tn, m), lambda i: (i, 0)),
        out_shape=jax.ShapeDtypeStruct((n, m), jnp.float32),
    )(x, w, b.reshape(1, -1))


def _tc_combine(h, parts, degp, w_h, w_a, w_b, m_w, m_b):
    tn = 1000

    def body(h_ref, p_ref, d_ref, wh_ref, wa_ref, wb_ref, mw_ref, mb_ref,
             hn_ref, o2_ref):
        deg = d_ref[0, :, :1] + d_ref[1, :, :1]
        agg = (p_ref[0] + p_ref[1]) / jnp.maximum(deg, 1.0)
        z = jax.nn.relu(
            jnp.dot(h_ref[...], wh_ref[...],
                    preferred_element_type=jnp.float32,
                    precision=lax.Precision.HIGHEST)
            + jnp.dot(agg, wa_ref[...],
                      preferred_element_type=jnp.float32,
                      precision=lax.Precision.HIGHEST)
            + wb_ref[...])
        nrm = jnp.sqrt(jnp.sum(z * z, axis=-1, keepdims=True))
        hn = z / jnp.maximum(nrm, 1e-6)
        hn_ref[...] = hn
        o2_ref[...] = jax.nn.relu(
            jnp.dot(hn, mw_ref[...],
                    preferred_element_type=jnp.float32,
                    precision=lax.Precision.HIGHEST) + mb_ref[...])

    return pl.pallas_call(
        body,
        grid=(N // tn,),
        in_specs=[
            pl.BlockSpec((tn, D), lambda i: (i, 0)),
            pl.BlockSpec((2, tn, H), lambda i: (0, i, 0)),
            pl.BlockSpec((2, tn, 1), lambda i: (0, i, 0)),
            pl.BlockSpec((D, D), lambda i: (0, 0)),
            pl.BlockSpec((H, D), lambda i: (0, 0)),
            pl.BlockSpec((1, D), lambda i: (0, 0)),
            pl.BlockSpec((D, D), lambda i: (0, 0)),
            pl.BlockSpec((1, D), lambda i: (0, 0)),
        ],
        out_specs=[
            pl.BlockSpec((tn, D), lambda i: (i, 0)),
            pl.BlockSpec((tn, D), lambda i: (i, 0)),
        ],
        out_shape=[
            jax.ShapeDtypeStruct((N, D), jnp.float32),
            jax.ShapeDtypeStruct((N, D), jnp.float32),
        ],
    )(h, parts, degp, w_h, w_a, w_b.reshape(1, -1), m_w, m_b.reshape(1, -1))


def kernel(x, edge_index, Q0_w, Q0_b, W0_w, W0_b, Q1_w, Q1_b, W1_w, W1_b, G_w, G_b):
    pad = jnp.zeros((8 * CH,), jnp.int32)
    src = jnp.concatenate([edge_index[0], pad])
    dst = jnp.concatenate([edge_index[1], pad])

    m0 = _tc_linear_relu(x, Q0_w, Q0_b)
    parts0, degp = _sc_aggregate(m0, src, dst, with_deg=True)
    degp = degp[:, :N].reshape(2, N, 1)

    h1, m1 = _tc_combine(x, parts0, degp, W0_w[:D], W0_w[D:], W0_b, Q1_w, Q1_b)

    parts1 = _sc_aggregate(m1, src, dst, with_deg=False)[0]
    _, out = _tc_combine(h1, parts1, degp, W1_w[:D], W1_w[D:], W1_b, G_w, G_b)
    return out
